# baseline (speedup 1.0000x reference)
import functools

import jax
import jax.numpy as jnp
from jax import lax
from jax.experimental import pallas as pl
from jax.experimental.pallas import tpu as pltpu
from jax.experimental.pallas import tpu_sc as plsc

_VOCAB = 100000
_HIDDEN = 768
_MAX_POS = 2048
_BATCH = 4
_SEQ = 2048

_L = 16
_NV = _HIDDEN // _L
_NW = 32
_S_PER_W = _SEQ // _NW
_CH = 16
_NBUF = 4
_NCHUNK = (_S_PER_W // _CH) * _BATCH
_NGRP = _NCHUNK // _NBUF
_CG = 8
_NCG = _NV // _CG
_INV_H = 1.0 / _HIDDEN
_EPS = 1e-12


def _lane_sum(v):
    dnums = lax.GatherDimensionNumbers(
        offset_dims=(), collapsed_slice_dims=(0,), start_index_map=(0,))
    for sh in (8, 4, 2, 1):
        idx = lax.iota(jnp.int32, _L) ^ sh
        v = v + lax.gather(v, idx[:, None], dnums, (1,),
                           mode=lax.GatherScatterMode.PROMISE_IN_BOUNDS)
    return v


def _rsqrt_vec(v):
    i = lax.bitcast_convert_type(v, jnp.int32)
    i = jnp.full((_L,), 0x5F3759DF, jnp.int32) - lax.shift_right_logical(
        i, jnp.full((_L,), 1, jnp.int32))
    y = lax.bitcast_convert_type(i, jnp.float32)
    half_v = v * 0.5
    for _ in range(3):
        y = y * (1.5 - half_v * y * y)
    return y


def _emb_body(x_hbm, pos_hbm, gamma_hbm, beta_hbm, tok_hbm, out_hbm,
              idx_v, pos_v, tok0_v, tok1_v, tok2_v, tok3_v,
              gamma_v, beta_v, mean_v, rstd_v,
              gsem0, gsem1, gsem2, gsem3,
              ssem0, ssem1, ssem2, ssem3, stage_sem):
    nc = 2
    wid = lax.axis_index("s") * nc + lax.axis_index("c")
    s0w = wid * _S_PER_W

    bufs = (tok0_v, tok1_v, tok2_v, tok3_v)
    gsems = (gsem0, gsem1, gsem2, gsem3)
    ssems = (ssem0, ssem1, ssem2, ssem3)

    stages = [(gamma_hbm, gamma_v), (beta_hbm, beta_v),
              (pos_hbm.at[pl.ds(s0w, _S_PER_W)], pos_v)]
    for c in range(_NCHUNK):
        q, b = c // _BATCH, c % _BATCH
        stages.append((x_hbm.at[b, pl.ds(s0w + q * _CH, _CH)], idx_v.at[c]))
    handles = [pltpu.async_copy(src, dst, stage_sem) for src, dst in stages]
    for h in handles:
        h.wait()

    def phase_a_rows(buf, pos_base, r):
        stats = []
        for u in range(2):
            acc0 = jnp.zeros((_L,), jnp.float32)
            acc1 = jnp.zeros((_L,), jnp.float32)
            q0 = jnp.zeros((_L,), jnp.float32)
            q1 = jnp.zeros((_L,), jnp.float32)
            for j in range(_NV):
                t = buf[r + u, pl.ds(j * _L, _L)] \
                    + pos_v[pos_base + r + u, pl.ds(j * _L, _L)]
                buf[r + u, pl.ds(j * _L, _L)] = t
                if j % 2 == 0:
                    acc0 = acc0 + t
                    q0 = q0 + t * t
                else:
                    acc1 = acc1 + t
                    q1 = q1 + t * t
            stats.append((acc0 + acc1, q0 + q1))
        for u, (acc, q) in enumerate(stats):
            mean = _lane_sum(acc) * _INV_H
            ex2 = _lane_sum(q) * _INV_H
            var = ex2 - mean * mean
            rstd = _rsqrt_vec(var + _EPS)
            mean_v[r + u] = mean
            rstd_v[r + u] = rstd

    def compute_chunk(buf, pos_base):
        def a_body(r2, _):
            phase_a_rows(buf, pos_base, r2 * 2)
            return 0
        lax.fori_loop(0, _CH // 2, a_body, 0, unroll=False)

        for cg in range(_NCG):
            gs = [gamma_v[pl.ds((cg * _CG + j) * _L, _L)] for j in range(_CG)]
            bs = [beta_v[pl.ds((cg * _CG + j) * _L, _L)] for j in range(_CG)]

            def b_body(r, _, cg=cg, gs=gs, bs=bs):
                m = mean_v[r]
                s = rstd_v[r]
                for j in range(_CG):
                    col = (cg * _CG + j) * _L
                    t = buf[r, pl.ds(col, _L)]
                    buf[r, pl.ds(col, _L)] = (t - m) * s * gs[j] + bs[j]
                return 0
            lax.fori_loop(0, _CH, b_body, 0, unroll=False)

    def gather(c, k):
        return pltpu.async_copy(tok_hbm.at[idx_v.at[c]], bufs[k], gsems[k])

    def wait_gather(c, k):
        pltpu.make_async_copy(tok_hbm.at[idx_v.at[c]], bufs[k],
                              gsems[k]).wait()

    def wait_store(k):
        pltpu.make_async_copy(bufs[k], out_hbm.at[0, pl.ds(0, _CH)],
                              ssems[k]).wait()

    for c in range(_NBUF - 1):
        gather(c, c)

    def group(i, _):
        for k in range(_NBUF):
            c = _NBUF * i + k
            wait_gather(c, k)
            compute_chunk(bufs[k], i * _CH)
            if k == 0:
                @pl.when(i > 0)
                def _():
                    wait_store((k - 1) % _NBUF)
                gather(c + _NBUF - 1, (k - 1) % _NBUF)
            else:
                wait_store(k - 1)

                @pl.when(i < _NGRP - 1)
                def _():
                    gather(c + _NBUF - 1, k - 1)
            pltpu.async_copy(
                bufs[k], out_hbm.at[k, pl.ds(s0w + i * _CH, _CH)], ssems[k])
        return 0

    lax.fori_loop(0, _NGRP, group, 0, unroll=False)
    wait_store(_NBUF - 1)


@jax.jit
def kernel(x, token_table, pos_table, gamma, beta):
    mesh = plsc.VectorSubcoreMesh(core_axis_name="c", subcore_axis_name="s")
    run = functools.partial(
        pl.kernel,
        mesh=mesh,
        out_type=jax.ShapeDtypeStruct((_BATCH, _SEQ, _HIDDEN), jnp.float32),
        scratch_types=[
            pltpu.VMEM((_NCHUNK, _CH), jnp.int32),
            pltpu.VMEM((_S_PER_W, _HIDDEN), jnp.float32),
            pltpu.VMEM((_CH, _HIDDEN), jnp.float32),
            pltpu.VMEM((_CH, _HIDDEN), jnp.float32),
            pltpu.VMEM((_CH, _HIDDEN), jnp.float32),
            pltpu.VMEM((_CH, _HIDDEN), jnp.float32),
            pltpu.VMEM((_HIDDEN,), jnp.float32),
            pltpu.VMEM((_HIDDEN,), jnp.float32),
            pltpu.VMEM((_CH, _L), jnp.float32),
            pltpu.VMEM((_CH, _L), jnp.float32),
            pltpu.SemaphoreType.DMA,
            pltpu.SemaphoreType.DMA,
            pltpu.SemaphoreType.DMA,
            pltpu.SemaphoreType.DMA,
            pltpu.SemaphoreType.DMA,
            pltpu.SemaphoreType.DMA,
            pltpu.SemaphoreType.DMA,
            pltpu.SemaphoreType.DMA,
            pltpu.SemaphoreType.DMA,
        ],
    )(_emb_body)
    return run(x, pos_table, gamma, beta, token_table)
